# baseline (speedup 1.0000x reference)
import functools

import jax
import jax.numpy as jnp
from jax import lax
from jax.experimental import pallas as pl
from jax.experimental.pallas import tpu as pltpu
from jax.experimental.pallas import tpu_sc as plsc


_TILE = 512


def _add_body(x_ref, pos_ref, out_ref):
    out_ref[...] = x_ref[...] + pos_ref[...][None, :, :]


def _kernel_tc(x, pos_table):
    batch, seq_len, embed_dim = x.shape
    grid = (seq_len // _TILE,)
    return pl.pallas_call(
        _add_body,
        grid=grid,
        in_specs=[
            pl.BlockSpec((batch, _TILE, embed_dim), lambda i: (0, i, 0)),
            pl.BlockSpec((_TILE, embed_dim), lambda i: (i, 0)),
        ],
        out_specs=pl.BlockSpec((batch, _TILE, embed_dim), lambda i: (0, i, 0)),
        out_shape=jax.ShapeDtypeStruct(x.shape, x.dtype),
    )(x, pos_table[:seq_len])


_LANES = 16
_CHROWS = 8
_NBANK = 3


def _kernel_sc(x, pos_table):
    batch, seq_len, embed_dim = x.shape
    info = plsc.get_sparse_core_info()
    nc, ns = info.num_cores, info.num_subcores
    nw = nc * ns
    rpw = seq_len // nw
    nch = rpw // _CHROWS
    cols = embed_dim // _LANES
    col_shift = cols.bit_length() - 1
    mesh = plsc.VectorSubcoreMesh(core_axis_name="c", subcore_axis_name="s")

    @functools.partial(
        pl.kernel,
        mesh=mesh,
        out_type=jax.ShapeDtypeStruct(x.shape, x.dtype),
        scratch_types=[
            pltpu.VMEM((_NBANK, batch, _CHROWS, embed_dim), jnp.float32),
            pltpu.VMEM((_NBANK, _CHROWS, embed_dim), jnp.float32),
        ]
        + [pltpu.SemaphoreType.DMA] * (2 * _NBANK * batch + _NBANK),
    )
    def sc_add(x_hbm, pos_hbm, out_hbm, x_s, pos_s, *sems):
        in_sems = [list(sems[k * batch : (k + 1) * batch]) for k in range(_NBANK)]
        out_sems = [
            list(sems[(_NBANK + k) * batch : (_NBANK + k + 1) * batch])
            for k in range(_NBANK)
        ]
        pos_sems = list(sems[2 * _NBANK * batch :])
        wid = lax.axis_index("s") * nc + lax.axis_index("c")
        base = wid * rpw

        def wait_x(slot_ref, sem):
            pltpu.make_async_copy(x_hbm.at[0, pl.ds(0, _CHROWS)], slot_ref, sem).wait()

        def wait_pos(slot_ref, sem):
            pltpu.make_async_copy(pos_hbm.at[pl.ds(0, _CHROWS)], slot_ref, sem).wait()

        def fire_pos(c_off, bank):
            pltpu.async_copy(
                pos_hbm.at[pl.ds(c_off, _CHROWS)], pos_s.at[bank], pos_sems[bank]
            )

        def fire_x(b, c_off, bank):
            pltpu.async_copy(
                x_hbm.at[b, pl.ds(c_off, _CHROWS)], x_s.at[bank, b], in_sems[bank][b]
            )

        def emit_chunk(sp, off, drain_nxt, fire_nxt):
            nxt = (sp + 2) % _NBANK
            wait_pos(pos_s.at[sp], pos_sems[sp])
            for b in range(batch):
                wait_x(x_s.at[sp, b], in_sems[sp][b])

                @plsc.parallel_loop(0, _CHROWS * cols, unroll=8)
                def _(g, b=b, sp=sp):
                    i = g >> col_shift
                    sl = pl.ds((g & (cols - 1)) * _LANES, _LANES)
                    plsc.addupdate(x_s.at[sp, b, i, sl], pos_s[sp, i, sl])

                pltpu.async_copy(
                    x_s.at[sp, b], out_hbm.at[b, pl.ds(off, _CHROWS)], out_sems[sp][b]
                )
                if fire_nxt:
                    if b == 0:
                        fire_pos(off + 2 * _CHROWS, nxt)
                    if drain_nxt:
                        wait_x(x_s.at[nxt, b], out_sems[nxt][b])
                    fire_x(b, off + 2 * _CHROWS, nxt)

        fire_pos(base, 0)
        fire_pos(base + _CHROWS, 1)
        for b in range(batch):
            fire_x(b, base, 0)
            fire_x(b, base + _CHROWS, 1)

        emit_chunk(0, base, drain_nxt=False, fire_nxt=True)
        emit_chunk(1, base + _CHROWS, drain_nxt=True, fire_nxt=True)
        emit_chunk(2, base + 2 * _CHROWS, drain_nxt=True, fire_nxt=True)

        ntrip = (nch - _NBANK) // _NBANK

        def trip_body(p, carry):
            off = base + (_NBANK * p + _NBANK) * _CHROWS
            for sp in range(_NBANK):
                emit_chunk(sp, off + sp * _CHROWS, drain_nxt=True, fire_nxt=True)
            return carry

        lax.fori_loop(0, ntrip, trip_body, 0)

        for c in range(_NBANK + _NBANK * ntrip, nch):
            live = c + 2 < nch
            emit_chunk(
                c % _NBANK, base + c * _CHROWS, drain_nxt=live, fire_nxt=live
            )

        for c in range(nch - _NBANK, nch):
            for b in range(batch):
                wait_x(x_s.at[c % _NBANK, b], out_sems[c % _NBANK][b])

    return sc_add(x, pos_table[:seq_len])


def kernel(x, pos_table):
    return _kernel_sc(x, pos_table)
